# Initial kernel scaffold; baseline (speedup 1.0000x reference)
import jax
import jax.numpy as jnp
from jax.experimental import pallas as pl


def kernel(x, W_enc, W_dec, b_dec):
    raise NotImplementedError("write your pallas kernel here")


# speedup vs baseline: 36.7838x; 36.7838x over previous
import functools

import jax
import jax.numpy as jnp
from jax import lax
from jax.experimental import pallas as pl
from jax.experimental.pallas import tpu as pltpu
from jax.experimental.pallas import tpu_sc as plsc

ACT_DIM = 768
DICT_DIM = 30720
BATCH_N = 2048
K_TOTAL = 32 * BATCH_N
L1_C = 1e-4

_NW = 32
_TOT = BATCH_N * DICT_DIM
_SHARD = _TOT // _NW
_CH = 16384
_NCH = _SHARD // _CH
_NB1 = 32768
_NB2 = 65536

_TD_ENC = 1024
_TD_DEC = 512
_DEC_BOUNDARIES = (3, 11, 27, 59)


def _norm_body(x_ref, b_ref, xce_ref, mean_ref, std_ref):
    x = x_ref[...]
    m = jnp.mean(x, axis=1, keepdims=True)
    xc = x - m
    var = jnp.sum(xc * xc, axis=1, keepdims=True) * (1.0 / (ACT_DIM - 1))
    s = jnp.sqrt(var)
    xn = xc / (s + 1e-5)
    xce_ref[...] = xn - b_ref[0:1, :]
    mean_ref[...] = jnp.broadcast_to(m, (BATCH_N, 128))
    std_ref[...] = jnp.broadcast_to(s, (BATCH_N, 128))


_norm_call = pl.pallas_call(
    _norm_body,
    out_shape=[
        jax.ShapeDtypeStruct((BATCH_N, ACT_DIM), jnp.float32),
        jax.ShapeDtypeStruct((BATCH_N, 128), jnp.float32),
        jax.ShapeDtypeStruct((BATCH_N, 128), jnp.float32),
    ],
)


def _enc_body(xce_ref, w_ref, o_ref):
    acts = jnp.maximum(
        jnp.dot(xce_ref[...], w_ref[...], preferred_element_type=jnp.float32),
        0.0,
    )
    o_ref[...] = lax.bitcast_convert_type(acts, jnp.int32)


_enc_call = pl.pallas_call(
    _enc_body,
    grid=(DICT_DIM // _TD_ENC,),
    in_specs=[
        pl.BlockSpec((BATCH_N, ACT_DIM), lambda i: (0, 0)),
        pl.BlockSpec((ACT_DIM, _TD_ENC), lambda i: (0, i)),
    ],
    out_specs=pl.BlockSpec((BATCH_N, _TD_ENC), lambda i: (0, i)),
    out_shape=jax.ShapeDtypeStruct((BATCH_N, DICT_DIM), jnp.int32),
)


def _hist1_body(acts_hbm, out_hbm, hist_v, buf_v):
    wid = lax.axis_index("c") * 16 + lax.axis_index("s")
    zero16 = jnp.zeros((16,), jnp.int32)
    ones16 = jnp.ones((16,), jnp.int32)

    def zbody(i, c):
        hist_v[pl.ds(i * 16, 16)] = zero16
        return c

    lax.fori_loop(0, _NB1 // 16, zbody, 0)

    base = wid * _SHARD

    def chunk_body(c, carry):
        pltpu.sync_copy(acts_hbm.at[pl.ds(base + c * _CH, _CH)], buf_v)

        def ibody(j, c2):
            off = j * 64
            for u in range(4):
                bits = buf_v[pl.ds(off + u * 16, 16)]
                idx = lax.shift_right_logical(bits, 16)
                m = bits > 0
                plsc.addupdate_scatter(hist_v, [idx], ones16, mask=m)
            return c2

        lax.fori_loop(0, _CH // 64, ibody, 0)
        return carry

    lax.fori_loop(0, _NCH, chunk_body, 0)
    pltpu.sync_copy(hist_v, out_hbm.at[wid])


def _hist2_body(acts_hbm, p_hbm, out_hbm, hist_v, buf_v, p_v):
    wid = lax.axis_index("c") * 16 + lax.axis_index("s")
    zero16 = jnp.zeros((16,), jnp.int32)
    ones16 = jnp.ones((16,), jnp.int32)
    pltpu.sync_copy(p_hbm, p_v)
    b1v = p_v[...]

    def zbody(i, c):
        hist_v[pl.ds(i * 16, 16)] = zero16
        return c

    lax.fori_loop(0, _NB2 // 16, zbody, 0)

    base = wid * _SHARD

    def chunk_body(c, carry):
        pltpu.sync_copy(acts_hbm.at[pl.ds(base + c * _CH, _CH)], buf_v)

        def ibody(j, c2):
            off = j * 64
            for u in range(4):
                bits = buf_v[pl.ds(off + u * 16, 16)]
                hi = lax.shift_right_logical(bits, 16)
                m = (hi == b1v) & (bits > 0)
                idx = jnp.bitwise_and(bits, 65535)
                plsc.addupdate_scatter(hist_v, [idx], ones16, mask=m)
            return c2

        lax.fori_loop(0, _CH // 64, ibody, 0)
        return carry

    lax.fori_loop(0, _NCH, chunk_body, 0)
    pltpu.sync_copy(hist_v, out_hbm.at[wid])


@functools.cache
def _sc_calls():
    mesh = plsc.VectorSubcoreMesh(
        core_axis_name="c", subcore_axis_name="s", num_cores=2, num_subcores=16
    )
    params = pltpu.CompilerParams(needs_layout_passes=False)
    hist1 = pl.kernel(
        _hist1_body,
        out_type=jax.ShapeDtypeStruct((_NW, _NB1), jnp.int32),
        mesh=mesh,
        compiler_params=params,
        scratch_types=[
            pltpu.VMEM((_NB1,), jnp.int32),
            pltpu.VMEM((_CH,), jnp.int32),
        ],
    )
    hist2 = pl.kernel(
        _hist2_body,
        out_type=jax.ShapeDtypeStruct((_NW, _NB2), jnp.int32),
        mesh=mesh,
        compiler_params=params,
        scratch_types=[
            pltpu.VMEM((_NB2,), jnp.int32),
            pltpu.VMEM((_CH,), jnp.int32),
            pltpu.VMEM((16,), jnp.int32),
        ],
    )
    return hist1, hist2


def _search1_body(h_ref, o_ref):
    h = jnp.sum(h_ref[...], axis=0, keepdims=True)
    idx = lax.broadcasted_iota(jnp.int32, (1, _NB1), 1)
    count_pos = jnp.sum(h)

    def step(t, lohi):
        lo, hi = lohi
        mid = (lo + hi) // 2
        s = jnp.sum(jnp.where(idx >= mid, h, 0))
        good = s >= K_TOTAL
        return (jnp.where(good, mid, lo), jnp.where(good, hi, mid))

    lo, _hi = lax.fori_loop(0, 15, step, (jnp.int32(0), jnp.int32(_NB1)))
    cnt_above = jnp.sum(jnp.where(idx >= lo + 1, h, 0))
    lane = lax.broadcasted_iota(jnp.int32, (8, 128), 1)
    o_ref[...] = jnp.where(
        lane == 0, lo, jnp.where(lane == 1, cnt_above, count_pos)
    )


_search1_call = pl.pallas_call(
    _search1_body,
    out_shape=jax.ShapeDtypeStruct((8, 128), jnp.int32),
)


def _search2_body(h_ref, p_ref, o_ref):
    h = jnp.sum(h_ref[...], axis=0, keepdims=True)
    idx = lax.broadcasted_iota(jnp.int32, (1, _NB2), 1)
    b1 = p_ref[0, 0]
    cnt_above = p_ref[0, 1]
    count_pos = p_ref[0, 2]
    k2 = K_TOTAL - cnt_above

    def step(t, lohi):
        lo, hi = lohi
        mid = (lo + hi) // 2
        s = jnp.sum(jnp.where(idx >= mid, h, 0))
        good = s >= k2
        return (jnp.where(good, mid, lo), jnp.where(good, hi, mid))

    lo, _hi = lax.fori_loop(0, 16, step, (jnp.int32(0), jnp.int32(_NB2)))
    tbits = jnp.where(
        count_pos >= K_TOTAL, jnp.bitwise_or(lax.shift_left(b1, 16), lo), 0
    )
    tb = jnp.broadcast_to(tbits, (8, 128))
    o_ref[...] = lax.bitcast_convert_type(tb, jnp.float32)


_search2_call = pl.pallas_call(
    _search2_body,
    out_shape=jax.ShapeDtypeStruct((8, 128), jnp.float32),
)


def _dec_body(
    a_ref, w_ref, xce_ref, t_ref, b_ref, mean_ref, std_ref,
    sae_ref, atk_ref, scal_ref, recon_ref, acc_ref,
):
    i = pl.program_id(0)
    t = t_ref[0, 0]
    a = lax.bitcast_convert_type(a_ref[...], jnp.float32)
    ak = jnp.where(a >= t, a, 0.0)
    atk_ref[...] = ak

    @pl.when(i == 0)
    def _init():
        recon_ref[...] = jnp.zeros((BATCH_N, ACT_DIM), jnp.float32)
        xce = xce_ref[...]
        acc_ref[0] = jnp.mean(xce * xce)
        acc_ref[1] = 0.0

    recon_ref[...] += jnp.dot(ak, w_ref[...], preferred_element_type=jnp.float32)
    acc_ref[1] += jnp.sum(ak)

    @pl.when(
        (i == _DEC_BOUNDARIES[0])
        | (i == _DEC_BOUNDARIES[1])
        | (i == _DEC_BOUNDARIES[2])
        | (i == _DEC_BOUNDARIES[3])
    )
    def _checkpoint():
        diff = recon_ref[...] - xce_ref[...]
        acc_ref[0] += jnp.mean(diff * diff)

    @pl.when(i == _DEC_BOUNDARIES[3])
    def _final():
        mean_l2 = acc_ref[0] * 0.2
        l1_norm = acc_ref[1] * (1.0 / BATCH_N)
        loss = mean_l2 + L1_C * l1_norm
        recon_full = recon_ref[...] + b_ref[0:1, :]
        sae_ref[...] = recon_full * std_ref[:, 0:1] + mean_ref[:, 0:1]
        lane = lax.broadcasted_iota(jnp.int32, (8, 128), 1)
        scal_ref[...] = jnp.where(lane == 0, loss, mean_l2)


_dec_call = pl.pallas_call(
    _dec_body,
    grid=(DICT_DIM // _TD_DEC,),
    in_specs=[
        pl.BlockSpec((BATCH_N, _TD_DEC), lambda i: (0, i)),
        pl.BlockSpec((_TD_DEC, ACT_DIM), lambda i: (i, 0)),
        pl.BlockSpec((BATCH_N, ACT_DIM), lambda i: (0, 0)),
        pl.BlockSpec((8, 128), lambda i: (0, 0)),
        pl.BlockSpec((8, ACT_DIM), lambda i: (0, 0)),
        pl.BlockSpec((BATCH_N, 128), lambda i: (0, 0)),
        pl.BlockSpec((BATCH_N, 128), lambda i: (0, 0)),
    ],
    out_specs=[
        pl.BlockSpec((BATCH_N, ACT_DIM), lambda i: (0, 0)),
        pl.BlockSpec((BATCH_N, _TD_DEC), lambda i: (0, i)),
        pl.BlockSpec((8, 128), lambda i: (0, 0)),
    ],
    out_shape=[
        jax.ShapeDtypeStruct((BATCH_N, ACT_DIM), jnp.float32),
        jax.ShapeDtypeStruct((BATCH_N, DICT_DIM), jnp.float32),
        jax.ShapeDtypeStruct((8, 128), jnp.float32),
    ],
    scratch_shapes=[
        pltpu.VMEM((BATCH_N, ACT_DIM), jnp.float32),
        pltpu.SMEM((2,), jnp.float32),
    ],
)


def kernel(x, W_enc, W_dec, b_dec):
    b2d = jnp.broadcast_to(b_dec[None, :], (8, ACT_DIM))
    xce, mean128, std128 = _norm_call(x, b2d)
    acts = _enc_call(xce, W_enc)
    flat = acts.reshape(-1)
    hist1_call, hist2_call = _sc_calls()
    h1 = hist1_call(flat)
    p1 = _search1_call(h1)
    b1vec = jnp.broadcast_to(p1[0, 0], (16,))
    h2 = hist2_call(flat, b1vec)
    tarr = _search2_call(h2, p1)
    sae, atk, scal = _dec_call(acts, W_dec, xce, tarr, b2d, mean128, std128)
    return sae, atk, scal[0, 0], scal[0, 1]
